# baseline (speedup 1.0000x reference)
import jax
import jax.numpy as jnp
from jax import lax
from jax.experimental import pallas as pl
from jax.experimental.pallas import tpu as pltpu
from jax.experimental.pallas import tpu_sc as plsc

_B = 16384
_F = 26
_FIELD = 100000
_NC = 2
_NS = 16
_NW = _NC * _NS
_BPW = _B // _NW
_IPW = _F * _BPW
_L = 16
_JW = _BPW // _L


def _fm_body(xr, table, out, xt_v, rows_v, acc_v, s0):
    wid = lax.axis_index("s") * _NC + lax.axis_index("c")
    table1d = table.at[0]

    pltpu.sync_copy(xr.at[wid], xt_v)

    def fire(f, _):
        pltpu.async_copy(
            table1d.at[pl.ds(f * _FIELD, _FIELD)].at[
                xt_v.at[pl.ds(f * _BPW, _BPW)]
            ],
            rows_v.at[pl.ds(f * _BPW, _BPW)],
            s0,
        )
        return 0

    lax.fori_loop(0, _F, fire, 0)

    pltpu.make_async_copy(table1d.at[pl.ds(0, _IPW)], rows_v, s0).wait()

    def red(j, _):
        a0 = rows_v[pl.ds(j * _L, _L)]
        a1 = rows_v[pl.ds(_BPW + j * _L, _L)]
        for f in range(2, _F, 2):
            a0 = a0 + rows_v[pl.ds(f * _BPW + j * _L, _L)]
        for f in range(3, _F, 2):
            a1 = a1 + rows_v[pl.ds(f * _BPW + j * _L, _L)]
        acc_v[pl.ds(j * _L, _L)] = a0 + a1
        return 0

    lax.fori_loop(0, _JW, red, 0)

    pltpu.sync_copy(acc_v, out.at[pl.ds(wid * _BPW, _BPW)])


@jax.jit
def _fm(xr, table2d):
    mesh = plsc.VectorSubcoreMesh(
        core_axis_name="c", subcore_axis_name="s", num_cores=_NC
    )
    return pl.kernel(
        _fm_body,
        mesh=mesh,
        out_type=jax.ShapeDtypeStruct((_B,), jnp.float32),
        scratch_types=[
            pltpu.VMEM((_IPW,), jnp.int32),
            pltpu.VMEM((_IPW,), jnp.float32),
            pltpu.VMEM((_BPW,), jnp.float32),
            pltpu.SemaphoreType.DMA,
        ],
    )(xr, table2d)


def kernel(x, table):
    xprep = jnp.transpose(x.reshape(_NW, _BPW, _F), (0, 2, 1)).reshape(_NW, _IPW)
    out = _fm(xprep, table.reshape(1, _F * _FIELD))
    return out.reshape(_B, 1)
